# baseline (speedup 1.0000x reference)
import functools

import jax
import jax.numpy as jnp
from jax import lax
from jax.experimental import pallas as pl
from jax.experimental.pallas import tpu as pltpu
from jax.experimental.pallas import tpu_sc as plsc

N_ROWS = 16384
N_COMP = 64
NC = 2
NS = 16
L = 16
NW = NC * NS
SAMP_PER_W = N_ROWS // NW
NCHUNK = 4
CHUNK_S = SAMP_PER_W // NCHUNK
GROUPS = CHUNK_S // L
CHAINS = 4
CHAIN_W = N_COMP // CHAINS

_mesh = plsc.VectorSubcoreMesh(
    core_axis_name="c", subcore_axis_name="s", num_cores=NC, num_subcores=NS)


@functools.partial(
    pl.kernel,
    out_type=[
        jax.ShapeDtypeStruct((NW, N_COMP), jnp.float32),
        jax.ShapeDtypeStruct((NW, L), jnp.float32),
    ],
    mesh=_mesh,
    compiler_params=pltpu.CompilerParams(needs_layout_passes=False),
    scratch_types=[
        pltpu.VMEM((NCHUNK, N_COMP, CHUNK_S), jnp.float32),
        pltpu.VMEM((N_COMP,), jnp.float32),
        pltpu.VMEM((L,), jnp.float32),
        pltpu.SemaphoreType.DMA,
        pltpu.SemaphoreType.DMA,
        pltpu.SemaphoreType.DMA,
        pltpu.SemaphoreType.DMA,
    ],
)
def _sc_partials(zt_hbm, flags_hbm, conf_hbm, buf_v, flags_v, conf_v,
                 sem0, sem1, sem2, sem3):
    wid = lax.axis_index("s") * NC + lax.axis_index("c")
    base = wid * SAMP_PER_W
    sems = [sem0, sem1, sem2, sem3]

    copies = [
        pltpu.async_copy(
            zt_hbm.at[:, pl.ds(base + k * CHUNK_S, CHUNK_S)],
            buf_v.at[k], sems[k])
        for k in range(NCHUNK)
    ]

    iota = lax.iota(jnp.int32, L)
    zeros = jnp.zeros((L,), jnp.float32)
    ones = jnp.ones((L,), jnp.float32)
    for k in range(N_COMP // L):
        flags_v[pl.ds(k * L, L)] = zeros

    acc = zeros
    for k in range(NCHUNK):
        copies[k].wait()
        kb = k

        def group_body(g, acc):
            s0 = g * L
            ms, args = [], []
            for t in range(CHAINS):
                c0 = t * CHAIN_W
                m = buf_v[kb, c0, pl.ds(s0, L)]
                a = jnp.full((L,), c0, jnp.int32)
                for c in range(c0 + 1, c0 + CHAIN_W):
                    v = buf_v[kb, c, pl.ds(s0, L)]
                    gt = v > m
                    a = jnp.where(gt, jnp.int32(c), a)
                    m = jnp.maximum(m, v)
                ms.append(m)
                args.append(a)
            m, a = ms[0], args[0]
            for t in range(1, CHAINS):
                gt = ms[t] > m
                a = jnp.where(gt, args[t], a)
                m = jnp.maximum(m, ms[t])
            plsc.store_scatter(flags_v, [a], ones)
            return acc + m

        acc = lax.fori_loop(0, GROUPS, group_body, acc)

    conf_v[...] = acc
    pltpu.sync_copy(flags_v, flags_hbm.at[wid])
    pltpu.sync_copy(conf_v, conf_hbm.at[wid])


def _reduce_body(flags_ref, conf_ref, pop_ref, mean_ref):
    flags = flags_ref[...]
    colmax = jnp.max(flags, axis=0, keepdims=True)
    pop = jnp.sum(jnp.where(colmax > 0.0, 1.0, 0.0))
    mean = jnp.sum(conf_ref[...]) * (1.0 / N_ROWS)
    pop_ref[...] = jnp.full((1, 1), pop)
    mean_ref[...] = jnp.full((1, 1), mean)


_reduce = pl.pallas_call(
    _reduce_body,
    out_shape=[
        jax.ShapeDtypeStruct((1, 1), jnp.float32),
        jax.ShapeDtypeStruct((1, 1), jnp.float32),
    ],
)


def _zcopy_body(zin_ref, zout_ref, sem):
    pltpu.async_copy(zin_ref, zout_ref, sem).wait()


_zcopy = pl.pallas_call(
    _zcopy_body,
    in_specs=[pl.BlockSpec(memory_space=pl.ANY)],
    out_specs=pl.BlockSpec(memory_space=pl.ANY),
    out_shape=jax.ShapeDtypeStruct((N_ROWS, 128), jnp.float32),
    scratch_shapes=[pltpu.SemaphoreType.DMA],
)


def kernel(z, z_cat):
    flags, conf = _sc_partials(z_cat.T)
    z_out = _zcopy(z)
    pop, mean = _reduce(flags, conf)
    return (z_out, pop[0, 0], mean[0, 0])


# speedup vs baseline: 8.6936x; 8.6936x over previous
import functools

import jax
import jax.numpy as jnp
from jax import lax
from jax.experimental import pallas as pl
from jax.experimental.pallas import tpu as pltpu
from jax.experimental.pallas import tpu_sc as plsc

N_ROWS = 16384
N_COMP = 64
NC = 2
NS = 16
L = 16
NW = NC * NS
SAMP_PER_W = N_ROWS // NW
NCHUNK = 4
CHUNK_S = SAMP_PER_W // NCHUNK
GROUPS = CHUNK_S // L
CHAINS = 4
CHAIN_W = N_COMP // CHAINS

_mesh = plsc.VectorSubcoreMesh(
    core_axis_name="c", subcore_axis_name="s", num_cores=NC, num_subcores=NS)


@functools.partial(
    pl.kernel,
    out_type=[
        jax.ShapeDtypeStruct((NW, N_COMP), jnp.float32),
        jax.ShapeDtypeStruct((NW, L), jnp.float32),
    ],
    mesh=_mesh,
    compiler_params=pltpu.CompilerParams(needs_layout_passes=False),
    scratch_types=[
        pltpu.VMEM((NCHUNK, N_COMP, CHUNK_S), jnp.float32),
        pltpu.VMEM((N_COMP,), jnp.float32),
        pltpu.VMEM((L,), jnp.float32),
        pltpu.SemaphoreType.DMA,
        pltpu.SemaphoreType.DMA,
        pltpu.SemaphoreType.DMA,
        pltpu.SemaphoreType.DMA,
    ],
)
def _sc_partials(zt_hbm, flags_hbm, conf_hbm, buf_v, flags_v, conf_v,
                 sem0, sem1, sem2, sem3):
    wid = lax.axis_index("s") * NC + lax.axis_index("c")
    base = wid * SAMP_PER_W
    sems = [sem0, sem1, sem2, sem3]

    copies = [
        pltpu.async_copy(
            zt_hbm.at[:, pl.ds(base + k * CHUNK_S, CHUNK_S)],
            buf_v.at[k], sems[k])
        for k in range(NCHUNK)
    ]

    iota = lax.iota(jnp.int32, L)
    zeros = jnp.zeros((L,), jnp.float32)
    ones = jnp.ones((L,), jnp.float32)
    for k in range(N_COMP // L):
        flags_v[pl.ds(k * L, L)] = zeros

    acc = zeros
    for k in range(NCHUNK):
        copies[k].wait()
        kb = k

        def group_body(g, acc):
            s0 = g * L
            ms, args = [], []
            for t in range(CHAINS):
                c0 = t * CHAIN_W
                m = buf_v[kb, c0, pl.ds(s0, L)]
                a = jnp.full((L,), c0, jnp.int32)
                for c in range(c0 + 1, c0 + CHAIN_W):
                    v = buf_v[kb, c, pl.ds(s0, L)]
                    gt = v > m
                    a = jnp.where(gt, jnp.int32(c), a)
                    m = jnp.maximum(m, v)
                ms.append(m)
                args.append(a)
            m, a = ms[0], args[0]
            for t in range(1, CHAINS):
                gt = ms[t] > m
                a = jnp.where(gt, args[t], a)
                m = jnp.maximum(m, ms[t])
            plsc.store_scatter(flags_v, [a], ones)
            return acc + m

        acc = lax.fori_loop(0, GROUPS, group_body, acc)

    conf_v[...] = acc
    pltpu.sync_copy(flags_v, flags_hbm.at[wid])
    pltpu.sync_copy(conf_v, conf_hbm.at[wid])


def _reduce_body(flags_ref, conf_ref, pop_ref, mean_ref):
    flags = flags_ref[...]
    colmax = jnp.max(flags, axis=0, keepdims=True)
    pop = jnp.sum(jnp.where(colmax > 0.0, 1.0, 0.0))
    mean = jnp.sum(conf_ref[...]) * (1.0 / N_ROWS)
    pop_ref[...] = jnp.full((1, 1), pop)
    mean_ref[...] = jnp.full((1, 1), mean)


_reduce = pl.pallas_call(
    _reduce_body,
    out_shape=[
        jax.ShapeDtypeStruct((1, 1), jnp.float32),
        jax.ShapeDtypeStruct((1, 1), jnp.float32),
    ],
)


_ZBLK = 1024
_ZNBLK = N_ROWS // _ZBLK
_ZNBUF = 4


def _zcopy_body(zin_ref, zout_ref, buf, in_sems, out_sems):
    def start_in(k):
        return pltpu.async_copy(
            zin_ref.at[pl.ds(k * _ZBLK, _ZBLK)], buf.at[k % _ZNBUF],
            in_sems.at[k % _ZNBUF])

    def start_out(k):
        return pltpu.async_copy(
            buf.at[k % _ZNBUF], zout_ref.at[pl.ds(k * _ZBLK, _ZBLK)],
            out_sems.at[k % _ZNBUF])

    ins = [start_in(k) for k in range(_ZNBUF - 1)] + [None]
    outs = [None] * _ZNBUF
    for k in range(_ZNBLK):
        b = k % _ZNBUF
        ins[b].wait()
        nxt = k + _ZNBUF - 1
        if nxt < _ZNBLK:
            if outs[nxt % _ZNBUF] is not None:
                outs[nxt % _ZNBUF].wait()
            ins[nxt % _ZNBUF] = start_in(nxt)
        outs[b] = start_out(k)
    for k in range(_ZNBLK - _ZNBUF, _ZNBLK):
        outs[k % _ZNBUF].wait()


_zcopy = pl.pallas_call(
    _zcopy_body,
    in_specs=[pl.BlockSpec(memory_space=pl.ANY)],
    out_specs=pl.BlockSpec(memory_space=pl.ANY),
    out_shape=jax.ShapeDtypeStruct((N_ROWS, 128), jnp.float32),
    scratch_shapes=[
        pltpu.VMEM((_ZNBUF, _ZBLK, 128), jnp.float32),
        pltpu.SemaphoreType.DMA((_ZNBUF,)),
        pltpu.SemaphoreType.DMA((_ZNBUF,)),
    ],
)


def kernel(z, z_cat):
    flags, conf = _sc_partials(z_cat.T)
    z_out = _zcopy(z)
    pop, mean = _reduce(flags, conf)
    return (z_out, pop[0, 0], mean[0, 0])


# speedup vs baseline: 9.7364x; 1.1199x over previous
import functools

import jax
import jax.numpy as jnp
from jax import lax
from jax.experimental import pallas as pl
from jax.experimental.pallas import tpu as pltpu
from jax.experimental.pallas import tpu_sc as plsc

N_ROWS = 16384
N_COMP = 64
NC = 2
NS = 16
L = 16
NW = NC * NS
SAMP_PER_W = N_ROWS // NW
NCHUNK = 4
CHUNK_S = SAMP_PER_W // NCHUNK
GROUPS = CHUNK_S // L
CHAINS = 4
CHAIN_W = N_COMP // CHAINS

_mesh = plsc.VectorSubcoreMesh(
    core_axis_name="c", subcore_axis_name="s", num_cores=NC, num_subcores=NS)


@functools.partial(
    pl.kernel,
    out_type=[
        jax.ShapeDtypeStruct((NW, N_COMP), jnp.float32),
        jax.ShapeDtypeStruct((NW, L), jnp.float32),
    ],
    mesh=_mesh,
    compiler_params=pltpu.CompilerParams(needs_layout_passes=False),
    scratch_types=[
        pltpu.VMEM((NCHUNK, N_COMP, CHUNK_S), jnp.float32),
        pltpu.VMEM((N_COMP,), jnp.float32),
        pltpu.VMEM((L,), jnp.float32),
        pltpu.SemaphoreType.DMA,
        pltpu.SemaphoreType.DMA,
        pltpu.SemaphoreType.DMA,
        pltpu.SemaphoreType.DMA,
    ],
)
def _sc_partials(zt_hbm, flags_hbm, conf_hbm, buf_v, flags_v, conf_v,
                 sem0, sem1, sem2, sem3):
    wid = lax.axis_index("s") * NC + lax.axis_index("c")
    base = wid * SAMP_PER_W
    sems = [sem0, sem1, sem2, sem3]

    copies = [
        pltpu.async_copy(
            zt_hbm.at[:, pl.ds(base + k * CHUNK_S, CHUNK_S)],
            buf_v.at[k], sems[k])
        for k in range(NCHUNK)
    ]

    iota = lax.iota(jnp.int32, L)
    zeros = jnp.zeros((L,), jnp.float32)
    ones = jnp.ones((L,), jnp.float32)
    for k in range(N_COMP // L):
        flags_v[pl.ds(k * L, L)] = zeros

    acc = zeros
    for k in range(NCHUNK):
        copies[k].wait()
        kb = k

        def group_body(g, acc):
            s0 = g * L
            ms, args = [], []
            for t in range(CHAINS):
                c0 = t * CHAIN_W
                m = buf_v[kb, c0, pl.ds(s0, L)]
                a = jnp.full((L,), c0, jnp.int32)
                for c in range(c0 + 1, c0 + CHAIN_W):
                    v = buf_v[kb, c, pl.ds(s0, L)]
                    gt = v > m
                    a = jnp.where(gt, jnp.int32(c), a)
                    m = jnp.maximum(m, v)
                ms.append(m)
                args.append(a)
            m, a = ms[0], args[0]
            for t in range(1, CHAINS):
                gt = ms[t] > m
                a = jnp.where(gt, args[t], a)
                m = jnp.maximum(m, ms[t])
            plsc.store_scatter(flags_v, [a], ones)
            return acc + m

        acc = lax.fori_loop(0, GROUPS, group_body, acc)

    conf_v[...] = acc
    pltpu.sync_copy(flags_v, flags_hbm.at[wid])
    pltpu.sync_copy(conf_v, conf_hbm.at[wid])


def _reduce_body(flags_ref, conf_ref, pop_ref, mean_ref):
    flags = flags_ref[...]
    colmax = jnp.max(flags, axis=0, keepdims=True)
    pop = jnp.sum(jnp.where(colmax > 0.0, 1.0, 0.0))
    mean = jnp.sum(conf_ref[...]) * (1.0 / N_ROWS)
    pop_ref[...] = jnp.full((1, 1), pop)
    mean_ref[...] = jnp.full((1, 1), mean)


_reduce = pl.pallas_call(
    _reduce_body,
    out_shape=[
        jax.ShapeDtypeStruct((1, 1), jnp.float32),
        jax.ShapeDtypeStruct((1, 1), jnp.float32),
    ],
)


_ZBLK = 1024
_ZNBLK = N_ROWS // _ZBLK


def _zcopy_body(zin_ref, zout_ref, buf, in_sems, out_sems):
    ins = [
        pltpu.async_copy(
            zin_ref.at[pl.ds(k * _ZBLK, _ZBLK)], buf.at[k], in_sems.at[k])
        for k in range(_ZNBLK)
    ]
    outs = []
    for k in range(_ZNBLK):
        ins[k].wait()
        outs.append(pltpu.async_copy(
            buf.at[k], zout_ref.at[pl.ds(k * _ZBLK, _ZBLK)], out_sems.at[k]))
    for o in outs:
        o.wait()


_zcopy = pl.pallas_call(
    _zcopy_body,
    in_specs=[pl.BlockSpec(memory_space=pl.ANY)],
    out_specs=pl.BlockSpec(memory_space=pl.ANY),
    out_shape=jax.ShapeDtypeStruct((N_ROWS, 128), jnp.float32),
    scratch_shapes=[
        pltpu.VMEM((_ZNBLK, _ZBLK, 128), jnp.float32),
        pltpu.SemaphoreType.DMA((_ZNBLK,)),
        pltpu.SemaphoreType.DMA((_ZNBLK,)),
    ],
)


def kernel(z, z_cat):
    flags, conf = _sc_partials(z_cat.T)
    z_out = _zcopy(z)
    pop, mean = _reduce(flags, conf)
    return (z_out, pop[0, 0], mean[0, 0])


# speedup vs baseline: 9.7979x; 1.0063x over previous
import functools

import jax
import jax.numpy as jnp
from jax import lax
from jax.experimental import pallas as pl
from jax.experimental.pallas import tpu as pltpu
from jax.experimental.pallas import tpu_sc as plsc

N_ROWS = 16384
N_COMP = 64
NC = 2
NS = 16
L = 16
NW = NC * NS
SAMP_PER_W = N_ROWS // NW
NCHUNK = 2
CHUNK_S = SAMP_PER_W // NCHUNK
GROUPS = CHUNK_S // L
CHAINS = 4
CHAIN_W = N_COMP // CHAINS

_mesh = plsc.VectorSubcoreMesh(
    core_axis_name="c", subcore_axis_name="s", num_cores=NC, num_subcores=NS)


@functools.partial(
    pl.kernel,
    out_type=[
        jax.ShapeDtypeStruct((NW, N_COMP), jnp.float32),
        jax.ShapeDtypeStruct((NW, L), jnp.float32),
    ],
    mesh=_mesh,
    compiler_params=pltpu.CompilerParams(needs_layout_passes=False),
    scratch_types=[
        pltpu.VMEM((NCHUNK, N_COMP, CHUNK_S), jnp.float32),
        pltpu.VMEM((N_COMP,), jnp.float32),
        pltpu.VMEM((L,), jnp.float32),
        pltpu.SemaphoreType.DMA,
        pltpu.SemaphoreType.DMA,
        pltpu.SemaphoreType.DMA,
        pltpu.SemaphoreType.DMA,
    ],
)
def _sc_partials(zt_hbm, flags_hbm, conf_hbm, buf_v, flags_v, conf_v,
                 sem0, sem1, sem2, sem3):
    wid = lax.axis_index("s") * NC + lax.axis_index("c")
    base = wid * SAMP_PER_W
    sems = [sem0, sem1, sem2, sem3]

    copies = [
        pltpu.async_copy(
            zt_hbm.at[:, pl.ds(base + k * CHUNK_S, CHUNK_S)],
            buf_v.at[k], sems[k])
        for k in range(NCHUNK)
    ]

    iota = lax.iota(jnp.int32, L)
    zeros = jnp.zeros((L,), jnp.float32)
    ones = jnp.ones((L,), jnp.float32)
    for k in range(N_COMP // L):
        flags_v[pl.ds(k * L, L)] = zeros

    acc = zeros
    for k in range(NCHUNK):
        copies[k].wait()
        kb = k

        def group_body(g, acc):
            s0 = g * L
            ms, args = [], []
            for t in range(CHAINS):
                c0 = t * CHAIN_W
                m = buf_v[kb, c0, pl.ds(s0, L)]
                a = jnp.full((L,), c0, jnp.int32)
                for c in range(c0 + 1, c0 + CHAIN_W):
                    v = buf_v[kb, c, pl.ds(s0, L)]
                    gt = v > m
                    a = jnp.where(gt, jnp.int32(c), a)
                    m = jnp.maximum(m, v)
                ms.append(m)
                args.append(a)
            m, a = ms[0], args[0]
            for t in range(1, CHAINS):
                gt = ms[t] > m
                a = jnp.where(gt, args[t], a)
                m = jnp.maximum(m, ms[t])
            plsc.store_scatter(flags_v, [a], ones)
            return acc + m

        acc = lax.fori_loop(0, GROUPS, group_body, acc)

    conf_v[...] = acc
    pltpu.sync_copy(flags_v, flags_hbm.at[wid])
    pltpu.sync_copy(conf_v, conf_hbm.at[wid])


def _reduce_body(flags_ref, conf_ref, pop_ref, mean_ref):
    flags = flags_ref[...]
    colmax = jnp.max(flags, axis=0, keepdims=True)
    pop = jnp.sum(jnp.where(colmax > 0.0, 1.0, 0.0))
    mean = jnp.sum(conf_ref[...]) * (1.0 / N_ROWS)
    pop_ref[...] = jnp.full((1, 1), pop)
    mean_ref[...] = jnp.full((1, 1), mean)


_reduce = pl.pallas_call(
    _reduce_body,
    out_shape=[
        jax.ShapeDtypeStruct((1, 1), jnp.float32),
        jax.ShapeDtypeStruct((1, 1), jnp.float32),
    ],
)


_ZBLK = 1024
_ZNBLK = N_ROWS // _ZBLK


def _zcopy_body(zin_ref, zout_ref, buf, in_sems, out_sems):
    ins = [
        pltpu.async_copy(
            zin_ref.at[pl.ds(k * _ZBLK, _ZBLK)], buf.at[k], in_sems.at[k])
        for k in range(_ZNBLK)
    ]
    outs = []
    for k in range(_ZNBLK):
        ins[k].wait()
        outs.append(pltpu.async_copy(
            buf.at[k], zout_ref.at[pl.ds(k * _ZBLK, _ZBLK)], out_sems.at[k]))
    for o in outs:
        o.wait()


_zcopy = pl.pallas_call(
    _zcopy_body,
    in_specs=[pl.BlockSpec(memory_space=pl.ANY)],
    out_specs=pl.BlockSpec(memory_space=pl.ANY),
    out_shape=jax.ShapeDtypeStruct((N_ROWS, 128), jnp.float32),
    scratch_shapes=[
        pltpu.VMEM((_ZNBLK, _ZBLK, 128), jnp.float32),
        pltpu.SemaphoreType.DMA((_ZNBLK,)),
        pltpu.SemaphoreType.DMA((_ZNBLK,)),
    ],
)


def kernel(z, z_cat):
    flags, conf = _sc_partials(z_cat.T)
    z_out = _zcopy(z)
    pop, mean = _reduce(flags, conf)
    return (z_out, pop[0, 0], mean[0, 0])


# speedup vs baseline: 10.0179x; 1.0225x over previous
import functools

import jax
import jax.numpy as jnp
from jax import lax
from jax.experimental import pallas as pl
from jax.experimental.pallas import tpu as pltpu
from jax.experimental.pallas import tpu_sc as plsc

N_ROWS = 16384
N_COMP = 64
NC = 2
NS = 16
L = 16
NW = NC * NS
SAMP_PER_W = N_ROWS // NW
NCHUNK = 2
CHUNK_S = SAMP_PER_W // NCHUNK
GROUPS = CHUNK_S // L
CHAINS = 4
CHAIN_W = N_COMP // CHAINS

_mesh = plsc.VectorSubcoreMesh(
    core_axis_name="c", subcore_axis_name="s", num_cores=NC, num_subcores=NS)


@functools.partial(
    pl.kernel,
    out_type=[
        jax.ShapeDtypeStruct((NW, N_COMP), jnp.float32),
        jax.ShapeDtypeStruct((NW, L), jnp.float32),
    ],
    mesh=_mesh,
    compiler_params=pltpu.CompilerParams(needs_layout_passes=False),
    scratch_types=[
        pltpu.VMEM((NCHUNK, N_COMP, CHUNK_S), jnp.float32),
        pltpu.VMEM((N_COMP,), jnp.float32),
        pltpu.VMEM((L,), jnp.float32),
        pltpu.SemaphoreType.DMA((NCHUNK * CHAINS,)),
    ],
)
def _sc_partials(zt_hbm, flags_hbm, conf_hbm, buf_v, flags_v, conf_v, sems):
    wid = lax.axis_index("s") * NC + lax.axis_index("c")
    base = wid * SAMP_PER_W

    copies = [
        [
            pltpu.async_copy(
                zt_hbm.at[pl.ds(t * CHAIN_W, CHAIN_W),
                          pl.ds(base + k * CHUNK_S, CHUNK_S)],
                buf_v.at[k, pl.ds(t * CHAIN_W, CHAIN_W)],
                sems.at[k * CHAINS + t])
            for t in range(CHAINS)
        ]
        for k in range(NCHUNK)
    ]

    iota = lax.iota(jnp.int32, L)
    zeros = jnp.zeros((L,), jnp.float32)
    ones = jnp.ones((L,), jnp.float32)
    for k in range(N_COMP // L):
        flags_v[pl.ds(k * L, L)] = zeros

    acc = zeros
    for k in range(NCHUNK):
        for c in copies[k]:
            c.wait()
        kb = k

        def group_body(g, acc):
            s0 = g * L
            ms, args = [], []
            for t in range(CHAINS):
                c0 = t * CHAIN_W
                m = buf_v[kb, c0, pl.ds(s0, L)]
                a = jnp.full((L,), c0, jnp.int32)
                for c in range(c0 + 1, c0 + CHAIN_W):
                    v = buf_v[kb, c, pl.ds(s0, L)]
                    gt = v > m
                    a = jnp.where(gt, jnp.int32(c), a)
                    m = jnp.maximum(m, v)
                ms.append(m)
                args.append(a)
            m, a = ms[0], args[0]
            for t in range(1, CHAINS):
                gt = ms[t] > m
                a = jnp.where(gt, args[t], a)
                m = jnp.maximum(m, ms[t])
            plsc.store_scatter(flags_v, [a], ones)
            return acc + m

        acc = lax.fori_loop(0, GROUPS, group_body, acc)

    conf_v[...] = acc
    pltpu.sync_copy(flags_v, flags_hbm.at[wid])
    pltpu.sync_copy(conf_v, conf_hbm.at[wid])


def _reduce_body(flags_ref, conf_ref, pop_ref, mean_ref):
    flags = flags_ref[...]
    colmax = jnp.max(flags, axis=0, keepdims=True)
    pop = jnp.sum(jnp.where(colmax > 0.0, 1.0, 0.0))
    mean = jnp.sum(conf_ref[...]) * (1.0 / N_ROWS)
    pop_ref[...] = jnp.full((1, 1), pop)
    mean_ref[...] = jnp.full((1, 1), mean)


_reduce = pl.pallas_call(
    _reduce_body,
    out_shape=[
        jax.ShapeDtypeStruct((1, 1), jnp.float32),
        jax.ShapeDtypeStruct((1, 1), jnp.float32),
    ],
)


_ZBLK = 1024
_ZNBLK = N_ROWS // _ZBLK


def _zcopy_body(zin_ref, zout_ref, buf, in_sems, out_sems):
    ins = [
        pltpu.async_copy(
            zin_ref.at[pl.ds(k * _ZBLK, _ZBLK)], buf.at[k], in_sems.at[k])
        for k in range(_ZNBLK)
    ]
    outs = []
    for k in range(_ZNBLK):
        ins[k].wait()
        outs.append(pltpu.async_copy(
            buf.at[k], zout_ref.at[pl.ds(k * _ZBLK, _ZBLK)], out_sems.at[k]))
    for o in outs:
        o.wait()


_zcopy = pl.pallas_call(
    _zcopy_body,
    in_specs=[pl.BlockSpec(memory_space=pl.ANY)],
    out_specs=pl.BlockSpec(memory_space=pl.ANY),
    out_shape=jax.ShapeDtypeStruct((N_ROWS, 128), jnp.float32),
    scratch_shapes=[
        pltpu.VMEM((_ZNBLK, _ZBLK, 128), jnp.float32),
        pltpu.SemaphoreType.DMA((_ZNBLK,)),
        pltpu.SemaphoreType.DMA((_ZNBLK,)),
    ],
)


def kernel(z, z_cat):
    flags, conf = _sc_partials(z_cat.T)
    z_out = _zcopy(z)
    pop, mean = _reduce(flags, conf)
    return (z_out, pop[0, 0], mean[0, 0])
